# baseline (speedup 1.0000x reference)
import functools

import jax
import jax.numpy as jnp
from jax import lax
from jax.experimental import pallas as pl
from jax.experimental.pallas import tpu as pltpu
from jax.experimental.pallas import tpu_sc as plsc

N = 320000
D = 128
S = 1024
TRASH = S
SACC = S + 8

NC = 2
NS = 16
NW = NC * NS

CH = 128
G = 16
NG = CH // G
NCHUNK = N // CH
NQ = NCHUNK // 4
QBASE = NQ // NW
QEXTRA = NQ - QBASE * NW
SVPAD = 40064
ROWS_PER_TILE = S // NS

RES_GROUPS = 1024
RES_ROWS = RES_GROUPS * G
RES_PER_TILE = RES_ROWS // NW
RES_BATCH = RES_PER_TILE // 16

_mesh = plsc.VectorSubcoreMesh(core_axis_name="c", subcore_axis_name="s")


@functools.partial(
    pl.kernel,
    mesh=_mesh,
    out_type=jax.ShapeDtypeStruct((NC, S, D), jnp.float32),
    scratch_types=[
        pltpu.VMEM((CH, D), jnp.float32),
        pltpu.VMEM((CH, D), jnp.float32),
        pltpu.VMEM((16, D), jnp.float32),
        pltpu.VMEM((16, D), jnp.float32),
        pltpu.VMEM((80 * 16,), jnp.int32),
        pltpu.VMEM((RES_PER_TILE,), jnp.int32),
        pltpu.VMEM((RES_PER_TILE,), jnp.int32),
        pltpu.VMEM((16, D), jnp.float32),
        pltpu.VMEM((16, D), jnp.float32),
        pltpu.VMEM((16, D), jnp.float32),
        pltpu.VMEM((16, D), jnp.float32),
        pltpu.VMEM_SHARED((SACC, D), jnp.float32),
        pltpu.SemaphoreType.DMA,
        pltpu.SemaphoreType.DMA,
        pltpu.SemaphoreType.DMA,
        pltpu.SemaphoreType.DMA,
        pltpu.SemaphoreType.DMA,
        pltpu.SemaphoreType.DMA,
        pltpu.SemaphoreType.DMA,
        pltpu.SemaphoreType.DMA,
        pltpu.SemaphoreType.DMA,
        pltpu.SemaphoreType.DMA,
        pltpu.SemaphoreType.DMA,
        pltpu.SemaphoreType.DMA,
    ],
)
def _sc_segsum(x_hbm, sv_hbm, rows_hbm, tgt_hbm, zs_hbm, sums_out,
               buf0, buf1, red0, red1, svbuf, rbuf, tbuf,
               st0, st1, st2, st3, acc,
               g0, g1, p0, p1, r0s, r1s, r2s, r3s, s0, s1, s2, s3):
    c = lax.axis_index("c")
    s = lax.axis_index("s")
    wid = s * NC + c

    bufs = (buf0, buf1)
    reds = (red0, red1)
    gsems = (g0, g1)
    psems = (p0, p1)
    stages = (st0, st1, st2, st3)
    rsems = (r0s, r1s, r2s, r3s)
    ssems = (s0, s1, s2, s3)

    nq = QBASE + jnp.where(wid < QEXTRA, 1, 0)
    q0 = wid * QBASE + jnp.minimum(wid, QEXTRA)
    c0 = q0 * 4
    ntot = nq * 4

    r0 = s * ROWS_PER_TILE
    pltpu.sync_copy(zs_hbm.at[pl.ds(r0, ROWS_PER_TILE)],
                    acc.at[pl.ds(r0, ROWS_PER_TILE)])
    pltpu.sync_copy(sv_hbm.at[pl.ds(q0 * 64, 80 * 16)], svbuf)
    rbase = wid * RES_PER_TILE
    pltpu.sync_copy(rows_hbm.at[pl.ds(rbase, RES_PER_TILE)], rbuf)
    pltpu.sync_copy(tgt_hbm.at[pl.ds(rbase, RES_PER_TILE)], tbuf)
    plsc.subcore_barrier()

    def gather_start(cid, e):
        base = pl.multiple_of(cid * CH, 8)
        pltpu.async_copy(x_hbm.at[pl.ds(base, CH)], bufs[e], gsems[e])

    def gather_wait(cid, e):
        base = pl.multiple_of(cid * CH, 8)
        pltpu.make_async_copy(x_hbm.at[pl.ds(base, CH)], bufs[e],
                              gsems[e]).wait()

    def psem_drain(e):
        pltpu.make_async_copy(zs_hbm.at[pl.ds(0, 16)], reds[e], psems[e]).wait()

    def pair_body(p, carry):
        for e in range(2):
            l = p * 2 + e
            cid = c0 + l

            gather_wait(cid, e)

            @pl.when(p >= 1)
            def _():
                psem_drain(e)

            for u in range(NG):
                gacc = [bufs[e][u * G, pl.ds(d * 16, 16)] for d in range(8)]
                for r in range(1, G):
                    for d in range(8):
                        gacc[d] = gacc[d] + bufs[e][u * G + r,
                                                    pl.ds(d * 16, 16)]
                for d in range(8):
                    reds[e][u, pl.ds(d * 16, 16)] = gacc[d]

            ivec = svbuf[pl.ds(l * 16, 16)]
            pltpu.async_copy(reds[e], acc.at[ivec], psems[e], add=True)

            @pl.when(l + 2 < ntot)
            def _():
                gather_start(cid + 2, e)
        return carry

    gather_start(c0, 0)
    gather_start(c0 + 1, 1)
    lax.fori_loop(0, nq * 2, pair_body, 0)
    psem_drain(0)
    psem_drain(1)

    def rgather_start(k, b):
        rvec = rbuf[pl.ds(k * 16, 16)]
        pltpu.async_copy(x_hbm.at[rvec], stages[b], rsems[b])

    def rgather_wait(k, b):
        rvec = rbuf[pl.ds(k * 16, 16)]
        pltpu.make_async_copy(x_hbm.at[rvec], stages[b], rsems[b]).wait()

    def ssem_drain(b):
        pltpu.make_async_copy(zs_hbm.at[pl.ds(0, 16)], stages[b],
                              ssems[b]).wait()

    for b in range(4):
        rgather_start(b, b)

    def res_body(rq, carry):
        for b in range(4):
            k = rq * 4 + b
            rgather_wait(k, b)
            tvec = tbuf[pl.ds(k * 16, 16)]
            pltpu.async_copy(stages[b], acc.at[tvec], ssems[b], add=True)

            @pl.when(k + 4 < RES_BATCH)
            def _():
                ssem_drain(b)
                rgather_start(k + 4, b)
        return carry

    lax.fori_loop(0, RES_BATCH // 4, res_body, 0)
    for b in range(4):
        ssem_drain(b)

    plsc.subcore_barrier()
    pltpu.sync_copy(acc.at[pl.ds(r0, ROWS_PER_TILE)],
                    sums_out.at[c, pl.ds(r0, ROWS_PER_TILE)])


def _tc_head(sums_ref, cnt_ref, g1_ref, be1_ref, w1_ref, b1_ref,
             w2_ref, b2_ref, g2_ref, be2_ref, out_ref):
    sums = sums_ref[0, :, :] + sums_ref[1, :, :]
    cnt = jnp.maximum(cnt_ref[:, :], 1.0)
    h = sums / cnt

    def layer_norm(v, gamma, beta):
        mean = jnp.mean(v, axis=-1, keepdims=True)
        var = jnp.var(v, axis=-1, keepdims=True)
        return (v - mean) * lax.rsqrt(var + 1e-5) * gamma + beta

    h = layer_norm(h, g1_ref[0:1, :], be1_ref[0:1, :])
    y = lax.dot_general(h, w1_ref[:, :], (((1,), (1,)), ((), ())),
                        preferred_element_type=jnp.float32,
                        precision=lax.Precision.HIGHEST) + b1_ref[0:1, :]
    y = jnp.where(y > 0, y, jnp.exp(jnp.minimum(y, 0.0)) - 1.0)
    y = lax.dot_general(y, w2_ref[:, :], (((1,), (1,)), ((), ())),
                        preferred_element_type=jnp.float32,
                        precision=lax.Precision.HIGHEST) + b2_ref[0:1, :]
    y = y + h
    out_ref[:, :] = layer_norm(y, g2_ref[0:1, :], be2_ref[0:1, :])


_tc_head_call = pl.pallas_call(
    _tc_head,
    out_shape=jax.ShapeDtypeStruct((S, D), jnp.float32),
)


@jax.jit
def kernel(x, graph_index, gamma1, beta1, W1, b1, W2, b2, gamma2, beta2):
    idx = graph_index.astype(jnp.int32)
    zeros_s = jnp.zeros((S, D), jnp.float32)

    g3 = idx.reshape(NCHUNK, NG, G)
    gf = g3[:, :, 0]
    guni = gf == g3[:, :, G - 1]
    sv = jnp.concatenate(
        [jnp.where(guni, gf, TRASH),
         jnp.full((NCHUNK, 16 - NG), TRASH, jnp.int32)], axis=1).reshape(-1)
    sv = jnp.concatenate(
        [sv, jnp.full((SVPAD - NCHUNK * 16,), TRASH, jnp.int32)])

    rg = jnp.logical_not(guni).reshape(-1)
    nres = jnp.sum(rg)
    gidx = jnp.nonzero(rg, size=RES_GROUPS, fill_value=0)[0]
    valid = jnp.arange(RES_GROUPS) < nres
    rows = (gidx[:, None] * G + jnp.arange(G)[None, :]).astype(jnp.int32)
    tgt = jnp.where(valid[:, None], idx[rows], TRASH).astype(jnp.int32)
    rows = rows.reshape(-1)
    tgt = tgt.reshape(-1)

    ss = jnp.searchsorted(idx, jnp.arange(S + 1, dtype=jnp.int32))
    cnt2d = (ss[1:] - ss[:-1]).astype(jnp.float32).reshape(S, 1)

    sums = _sc_segsum(x, sv, rows, tgt, zeros_s)

    return _tc_head_call(
        sums, cnt2d,
        gamma1.reshape(1, D), beta1.reshape(1, D), W1, b1.reshape(1, D),
        W2, b2.reshape(1, D), gamma2.reshape(1, D), beta2.reshape(1, D))


# speedup vs baseline: 2.0415x; 2.0415x over previous
import functools

import jax
import jax.numpy as jnp
from jax import lax
from jax.experimental import pallas as pl
from jax.experimental.pallas import tpu as pltpu
from jax.experimental.pallas import tpu_sc as plsc

N = 320000
D = 128
S = 1024

NC = 2
NS = 16
NW = NC * NS

ROWS_PER_WORKER = N // NW
B = 400
NBW = ROWS_PER_WORKER // B
SUB = 80
NSUB = B // SUB
ROWS_PER_TILE = S // NS

_mesh = plsc.VectorSubcoreMesh(core_axis_name="c", subcore_axis_name="s")


@functools.partial(
    pl.kernel,
    mesh=_mesh,
    out_type=[
        jax.ShapeDtypeStruct((NC, S, D), jnp.float32),
        jax.ShapeDtypeStruct((NC, S, 16), jnp.float32),
    ],
    scratch_types=[
        pltpu.VMEM((B, D), jnp.float32),
        pltpu.VMEM((B, D), jnp.float32),
        pltpu.VMEM((NSUB, SUB), jnp.int32),
        pltpu.VMEM((NSUB, SUB), jnp.int32),
        pltpu.VMEM((SUB, 16), jnp.float32),
        pltpu.VMEM_SHARED((S, D), jnp.float32),
        pltpu.VMEM_SHARED((S, 16), jnp.float32),
        pltpu.SemaphoreType.DMA,
        pltpu.SemaphoreType.DMA,
        pltpu.SemaphoreType.DMA,
    ],
)
def _sc_segment_sum(x_hbm, idx_hbm, zs_hbm, zc_hbm, ones_hbm,
                    sums_out, counts_out,
                    rowbuf0, rowbuf1, idxbuf0, idxbuf1, onesbuf, acc, accc,
                    gsem0, gsem1, ssem):
    c = lax.axis_index("c")
    s = lax.axis_index("s")
    wid = s * NC + c
    blk0 = wid * NBW

    rowbufs = (rowbuf0, rowbuf1)
    idxbufs = (idxbuf0, idxbuf1)
    gsems = (gsem0, gsem1)

    r0 = s * ROWS_PER_TILE
    pltpu.sync_copy(zs_hbm.at[pl.ds(r0, ROWS_PER_TILE)], acc.at[pl.ds(r0, ROWS_PER_TILE)])
    pltpu.sync_copy(zc_hbm.at[pl.ds(r0, ROWS_PER_TILE)], accc.at[pl.ds(r0, ROWS_PER_TILE)])
    pltpu.sync_copy(ones_hbm, onesbuf)
    plsc.subcore_barrier()

    def row_base(j):
        return pl.multiple_of((blk0 + j) * B, 16)

    def gather_start(j, p):
        pltpu.async_copy(x_hbm.at[pl.ds(row_base(j), B)], rowbufs[p], gsems[p])
        pltpu.async_copy(idx_hbm.at[blk0 + j], idxbufs[p], gsems[p])

    def gather_wait(j, p):
        pltpu.make_async_copy(x_hbm.at[pl.ds(row_base(j), B)], rowbufs[p],
                              gsems[p]).wait()
        pltpu.make_async_copy(idx_hbm.at[blk0 + j], idxbufs[p],
                              gsems[p]).wait()

    def scatter_block(j, p):
        buf = rowbufs[p]
        idxb = idxbufs[p]
        cps = []
        for t in range(NSUB):
            cps.append(pltpu.async_copy(
                buf.at[pl.ds(t * SUB, SUB)],
                acc.at[idxb.at[t]], ssem, add=True))
            cps.append(pltpu.async_copy(
                onesbuf, accc.at[idxb.at[t]], ssem, add=True))
        for cp in cps:
            cp.wait()

    gather_start(0, 0)

    def pair_body(k2, carry):
        j0 = k2 * 2
        for p in range(2):
            j = j0 + p
            gather_wait(j, p)
            gather_start(j + 1, 1 - p)
            scatter_block(j, p)
        return carry

    lax.fori_loop(0, (NBW - 1) // 2, pair_body, 0)
    gather_wait(NBW - 1, 0)
    scatter_block(NBW - 1, 0)

    plsc.subcore_barrier()
    pltpu.sync_copy(acc.at[pl.ds(r0, ROWS_PER_TILE)],
                    sums_out.at[c, pl.ds(r0, ROWS_PER_TILE)])
    pltpu.sync_copy(accc.at[pl.ds(r0, ROWS_PER_TILE)],
                    counts_out.at[c, pl.ds(r0, ROWS_PER_TILE)])


def _tc_head(sums_ref, counts_ref, g1_ref, be1_ref, w1_ref, b1_ref,
             w2_ref, b2_ref, g2_ref, be2_ref, out_ref):
    sums = sums_ref[0, :, :] + sums_ref[1, :, :]
    counts = counts_ref[0, :, :] + counts_ref[1, :, :]
    cnt = jnp.maximum(counts[:, 0:1], 1.0)
    h = sums / cnt

    def layer_norm(v, gamma, beta):
        mean = jnp.mean(v, axis=-1, keepdims=True)
        var = jnp.var(v, axis=-1, keepdims=True)
        return (v - mean) * lax.rsqrt(var + 1e-5) * gamma + beta

    h = layer_norm(h, g1_ref[0:1, :], be1_ref[0:1, :])
    y = lax.dot_general(h, w1_ref[:, :], (((1,), (1,)), ((), ())),
                        preferred_element_type=jnp.float32,
                        precision=lax.Precision.HIGHEST) + b1_ref[0:1, :]
    y = jnp.where(y > 0, y, jnp.exp(jnp.minimum(y, 0.0)) - 1.0)
    y = lax.dot_general(y, w2_ref[:, :], (((1,), (1,)), ((), ())),
                        preferred_element_type=jnp.float32,
                        precision=lax.Precision.HIGHEST) + b2_ref[0:1, :]
    y = y + h
    out_ref[:, :] = layer_norm(y, g2_ref[0:1, :], be2_ref[0:1, :])


_tc_head_call = pl.pallas_call(
    _tc_head,
    out_shape=jax.ShapeDtypeStruct((S, D), jnp.float32),
)


@jax.jit
def kernel(x, graph_index, gamma1, beta1, W1, b1, W2, b2, gamma2, beta2):
    idx = graph_index.astype(jnp.int32).reshape(N // B, NSUB, SUB)
    zeros_s = jnp.zeros((S, D), jnp.float32)
    zeros_c = jnp.zeros((S, 16), jnp.float32)
    ones_b = jnp.ones((SUB, 16), jnp.float32)
    sums, counts = _sc_segment_sum(x, idx, zeros_s, zeros_c, ones_b)
    return _tc_head_call(
        sums, counts,
        gamma1.reshape(1, D), beta1.reshape(1, D), W1, b1.reshape(1, D),
        W2, b2.reshape(1, D), gamma2.reshape(1, D), beta2.reshape(1, D))


# speedup vs baseline: 2.1448x; 1.0506x over previous
import functools

import jax
import jax.numpy as jnp
from jax import lax
from jax.experimental import pallas as pl
from jax.experimental.pallas import tpu as pltpu
from jax.experimental.pallas import tpu_sc as plsc

N = 320000
D = 128
S = 1024

NC = 2
NS = 16
NW = NC * NS

ROWS_PER_WORKER = N // NW
B = 400
NBW = ROWS_PER_WORKER // B
SUB = 80
NSUB = B // SUB
ROWS_PER_TILE = S // NS

_mesh = plsc.VectorSubcoreMesh(core_axis_name="c", subcore_axis_name="s")


@functools.partial(
    pl.kernel,
    mesh=_mesh,
    out_type=jax.ShapeDtypeStruct((NC, S, D), jnp.float32),
    scratch_types=[
        pltpu.VMEM((B, D), jnp.float32),
        pltpu.VMEM((B, D), jnp.float32),
        pltpu.VMEM((NSUB, SUB), jnp.int32),
        pltpu.VMEM((NSUB, SUB), jnp.int32),
        pltpu.VMEM_SHARED((S, D), jnp.float32),
        pltpu.SemaphoreType.DMA,
        pltpu.SemaphoreType.DMA,
        pltpu.SemaphoreType.DMA,
    ],
)
def _sc_segment_sum(x_hbm, idx_hbm, zs_hbm,
                    sums_out,
                    rowbuf0, rowbuf1, idxbuf0, idxbuf1, acc,
                    gsem0, gsem1, ssem):
    c = lax.axis_index("c")
    s = lax.axis_index("s")
    wid = s * NC + c
    blk0 = wid * NBW

    rowbufs = (rowbuf0, rowbuf1)
    idxbufs = (idxbuf0, idxbuf1)
    gsems = (gsem0, gsem1)

    r0 = s * ROWS_PER_TILE
    pltpu.sync_copy(zs_hbm.at[pl.ds(r0, ROWS_PER_TILE)], acc.at[pl.ds(r0, ROWS_PER_TILE)])
    plsc.subcore_barrier()

    def row_base(j):
        return pl.multiple_of((blk0 + j) * B, 16)

    def gather_start(j, p):
        pltpu.async_copy(x_hbm.at[pl.ds(row_base(j), B)], rowbufs[p], gsems[p])
        pltpu.async_copy(idx_hbm.at[blk0 + j], idxbufs[p], gsems[p])

    def gather_wait(j, p):
        pltpu.make_async_copy(x_hbm.at[pl.ds(row_base(j), B)], rowbufs[p],
                              gsems[p]).wait()
        pltpu.make_async_copy(idx_hbm.at[blk0 + j], idxbufs[p],
                              gsems[p]).wait()

    def scatter_block(j, p):
        buf = rowbufs[p]
        idxb = idxbufs[p]
        cps = []
        for t in range(NSUB):
            cps.append(pltpu.async_copy(
                buf.at[pl.ds(t * SUB, SUB)],
                acc.at[idxb.at[t]], ssem, add=True))
        for cp in cps:
            cp.wait()

    gather_start(0, 0)

    def pair_body(k2, carry):
        j0 = k2 * 2
        for p in range(2):
            j = j0 + p
            gather_wait(j, p)
            gather_start(j + 1, 1 - p)
            scatter_block(j, p)
        return carry

    lax.fori_loop(0, (NBW - 1) // 2, pair_body, 0)
    gather_wait(NBW - 1, 0)
    scatter_block(NBW - 1, 0)

    plsc.subcore_barrier()
    pltpu.sync_copy(acc.at[pl.ds(r0, ROWS_PER_TILE)],
                    sums_out.at[c, pl.ds(r0, ROWS_PER_TILE)])


def _tc_head(sums_ref, cnt_ref, g1_ref, be1_ref, w1_ref, b1_ref,
             w2_ref, b2_ref, g2_ref, be2_ref, out_ref):
    sums = sums_ref[0, :, :] + sums_ref[1, :, :]
    cnt = jnp.maximum(cnt_ref[:, :], 1.0)
    h = sums / cnt

    def layer_norm(v, gamma, beta):
        mean = jnp.mean(v, axis=-1, keepdims=True)
        var = jnp.var(v, axis=-1, keepdims=True)
        return (v - mean) * lax.rsqrt(var + 1e-5) * gamma + beta

    h = layer_norm(h, g1_ref[0:1, :], be1_ref[0:1, :])
    y = lax.dot_general(h, w1_ref[:, :], (((1,), (1,)), ((), ())),
                        preferred_element_type=jnp.float32,
                        precision=lax.Precision.HIGHEST) + b1_ref[0:1, :]
    y = jnp.where(y > 0, y, jnp.exp(jnp.minimum(y, 0.0)) - 1.0)
    y = lax.dot_general(y, w2_ref[:, :], (((1,), (1,)), ((), ())),
                        preferred_element_type=jnp.float32,
                        precision=lax.Precision.HIGHEST) + b2_ref[0:1, :]
    y = y + h
    out_ref[:, :] = layer_norm(y, g2_ref[0:1, :], be2_ref[0:1, :])


_tc_head_call = pl.pallas_call(
    _tc_head,
    out_shape=jax.ShapeDtypeStruct((S, D), jnp.float32),
)


@jax.jit
def kernel(x, graph_index, gamma1, beta1, W1, b1, W2, b2, gamma2, beta2):
    idx1 = graph_index.astype(jnp.int32)
    idx = idx1.reshape(N // B, NSUB, SUB)
    zeros_s = jnp.zeros((S, D), jnp.float32)
    ss = jnp.searchsorted(idx1, jnp.arange(S + 1, dtype=jnp.int32))
    cnt2d = (ss[1:] - ss[:-1]).astype(jnp.float32).reshape(S, 1)
    sums = _sc_segment_sum(x, idx, zeros_s)
    return _tc_head_call(
        sums, cnt2d,
        gamma1.reshape(1, D), beta1.reshape(1, D), W1, b1.reshape(1, D),
        W2, b2.reshape(1, D), gamma2.reshape(1, D), beta2.reshape(1, D))
